# baseline (speedup 1.0000x reference)
import functools

import jax
import jax.numpy as jnp
from jax import lax
from jax.experimental import pallas as pl
from jax.experimental.pallas import tpu as pltpu
from jax.experimental.pallas import tpu_sc as plsc

N_GENDER = 2
N_AGE = 7
N_OCC = 21
N_ROWS = N_GENDER + N_AGE + N_OCC
DIM = 64
BATCH = 16384
ROWS = BATCH * 3
CHUNK = 128
LANES = 16

_info = plsc.get_sparse_core_info()
_NC, _NS = _info.num_cores, _info.num_subcores
NW = _NC * _NS
PER_W = ROWS // NW
NCHUNK = PER_W // CHUNK
HALF = NCHUNK // 2


@functools.partial(
    pl.kernel,
    out_type=jax.ShapeDtypeStruct((ROWS, DIM), jnp.float32),
    mesh=plsc.VectorSubcoreMesh(core_axis_name="c", subcore_axis_name="s"),
    scratch_types=[
        pltpu.VMEM((NCHUNK, CHUNK), jnp.int32),
        pltpu.VMEM((N_ROWS, DIM), jnp.float32),
        pltpu.VMEM((PER_W, DIM), jnp.float32),
        pltpu.HBM((NW * N_ROWS, DIM), jnp.float32),
        pltpu.SemaphoreType.DMA,
        pltpu.SemaphoreType.DMA,
        pltpu.SemaphoreType.DMA,
    ],
    compiler_params=pltpu.CompilerParams(use_tc_tiling_on_sc=False),
)
def _embed(x1_hbm, table_hbm, out_hbm, idx_v, table_v, rows_v, trep_hbm,
           sem_a, sem_b, sem_o):
    wid = lax.axis_index("s") * _NC + lax.axis_index("c")
    base = wid * PER_W
    rowbase = wid * N_ROWS

    pltpu.sync_copy(x1_hbm.at[wid], idx_v)
    pltpu.sync_copy(table_hbm, table_v)
    pltpu.sync_copy(table_v, trep_hbm.at[pl.ds(rowbase, N_ROWS)])

    lane = lax.iota(jnp.int32, LANES)
    pats = []
    for m in range(3):
        r = lax.rem(lane + m, 3)
        pats.append(
            jnp.where(r == 0, 0, jnp.where(r == 1, N_GENDER, N_GENDER + N_AGE))
            + rowbase)
    for j in range(NCHUNK):
        for k in range(CHUNK // LANES):
            p = j * CHUNK + k * LANES
            idx_v[j, pl.ds(k * LANES, LANES)] = (
                idx_v[j, pl.ds(k * LANES, LANES)] + pats[p % 3])

    gathers = [
        pltpu.async_copy(
            trep_hbm.at[idx_v.at[j]],
            rows_v.at[pl.ds(j * CHUNK, CHUNK)],
            sem_a if j < HALF else sem_b,
        )
        for j in range(NCHUNK)
    ]
    for j in range(HALF):
        gathers[j].wait()
    out_a = pltpu.async_copy(
        rows_v.at[pl.ds(0, HALF * CHUNK)],
        out_hbm.at[pl.ds(base, HALF * CHUNK)],
        sem_o,
    )
    for j in range(HALF, NCHUNK):
        gathers[j].wait()
    out_b = pltpu.async_copy(
        rows_v.at[pl.ds(HALF * CHUNK, PER_W - HALF * CHUNK)],
        out_hbm.at[pl.ds(base + HALF * CHUNK, PER_W - HALF * CHUNK)],
        sem_o,
    )
    out_a.wait()
    out_b.wait()


def kernel(x1, W_gender, W_age, W_occupation):
    x1r = x1.reshape(NW, NCHUNK, CHUNK)
    table = jnp.concatenate([W_gender, W_age, W_occupation], axis=0)
    out = _embed(x1r, table)
    return out.reshape(BATCH, 3 * DIM)
